# baseline (speedup 1.0000x reference)
import functools

import jax
import jax.numpy as jnp
from jax.experimental import pallas as pl
from jax.experimental.pallas import tpu as pltpu

TOPK = 70
TEMP = 0.05
Q = 4096
K_DB = 100000
D = 128
K_PAD = 100352
N_CHUNK = K_PAD // 128

BQ = 512
BK = 2048
CPB = BK // 128


def _norm_body(x_ref, o_ref):
    x = x_ref[...]
    n = jnp.sqrt(jnp.sum(x * x, axis=1, keepdims=True))
    o_ref[...] = x / jnp.maximum(n, 1e-12)


def _l2norm_rows(x, blk):
    r = x.shape[0]
    return pl.pallas_call(
        _norm_body,
        grid=(r // blk,),
        in_specs=[pl.BlockSpec((blk, D), lambda i: (i, 0))],
        out_specs=pl.BlockSpec((blk, D), lambda i: (i, 0)),
        out_shape=jax.ShapeDtypeStruct((r, D), jnp.float32),
    )(x)


def _sims_body(q_ref, e_ref, sims_ref, cm_ref):
    j = pl.program_id(1)
    s = jax.lax.dot_general(
        q_ref[...], e_ref[...], (((1,), (1,)), ((), ())),
        preferred_element_type=jnp.float32,
        precision=jax.lax.Precision.DEFAULT,
    )
    col = jax.lax.broadcasted_iota(jnp.int32, (BQ, BK), 1) + j * BK
    s = jnp.where(col < K_DB, s, -1e30)
    sims_ref[...] = s
    cm = jnp.stack(
        [jnp.max(s[:, c * 128:(c + 1) * 128], axis=1) for c in range(CPB)],
        axis=0)
    cm_ref[...] = cm


def _sims_call(qn, en_norm):
    return pl.pallas_call(
        _sims_body,
        grid=(Q // BQ, K_PAD // BK),
        in_specs=[
            pl.BlockSpec((BQ, D), lambda i, j: (i, 0)),
            pl.BlockSpec((BK, D), lambda i, j: (j, 0)),
        ],
        out_specs=[
            pl.BlockSpec((BQ, BK), lambda i, j: (i, j)),
            pl.BlockSpec((CPB, BQ), lambda i, j: (j, i)),
        ],
        out_shape=[
            jax.ShapeDtypeStruct((Q, K_PAD), jnp.float32),
            jax.ShapeDtypeStruct((N_CHUNK, Q), jnp.float32),
        ],
        compiler_params=pltpu.CompilerParams(
            dimension_semantics=("arbitrary", "arbitrary"),
        ),
    )(qn, en_norm)


def kernel(query, en_db, es_db):
    en_pad = jnp.pad(en_db, ((0, K_PAD - K_DB), (0, 0)))
    en_norm = _l2norm_rows(en_pad, 2048)
    qn = _l2norm_rows(query, 2048)
    sims, _cmT = _sims_call(qn, en_norm)
    topk_sims, topk_idx = jax.lax.top_k(sims, TOPK)
    w = jax.nn.softmax(topk_sims / TEMP, axis=1)
    rows = es_db[topk_idx]
    return jnp.einsum('qk,qkd->qd', w, rows)


# speedup vs baseline: 16.7396x; 16.7396x over previous
import jax
import jax.numpy as jnp
from jax import lax
from jax.experimental import pallas as pl
from jax.experimental.pallas import tpu as pltpu
from jax.experimental.pallas import tpu_sc as plsc

TOPK = 70
TEMP = 0.05
Q = 4096
K_DB = 100000
D = 128
K_PAD = 100352
N_CHUNK = K_PAD // 128

BQ = 512
BK = 2048
CPB = BK // 128

NW = 32
RPW = Q // NW
GR = 16
NG = 96
NCAND = 256
NCBUF = NCAND + 16


def _norm_body(x_ref, o_ref):
    x = x_ref[...]
    n = jnp.sqrt(jnp.sum(x * x, axis=1, keepdims=True))
    o_ref[...] = x / jnp.maximum(n, 1e-12)


def _l2norm_rows(x, blk):
    r = x.shape[0]
    return pl.pallas_call(
        _norm_body,
        grid=(r // blk,),
        in_specs=[pl.BlockSpec((blk, D), lambda i: (i, 0))],
        out_specs=pl.BlockSpec((blk, D), lambda i: (i, 0)),
        out_shape=jax.ShapeDtypeStruct((r, D), jnp.float32),
    )(x)


def _sims_body(q_ref, e_ref, sims_ref, cm_ref):
    j = pl.program_id(1)
    s = lax.dot_general(
        q_ref[...], e_ref[...], (((1,), (1,)), ((), ())),
        preferred_element_type=jnp.float32,
        precision=lax.Precision.DEFAULT,
    )
    col = lax.broadcasted_iota(jnp.int32, (BQ, BK), 1) + j * BK
    s = jnp.where(col < K_DB, s, -1e30)
    sims_ref[...] = s
    cm = jnp.stack(
        [jnp.max(s[:, c * 128:(c + 1) * 128], axis=1) for c in range(CPB)],
        axis=0)
    cm_ref[...] = cm


def _sims_call(qn, en_norm):
    return pl.pallas_call(
        _sims_body,
        grid=(Q // BQ, K_PAD // BK),
        in_specs=[
            pl.BlockSpec((BQ, D), lambda i, j: (i, 0)),
            pl.BlockSpec((BK, D), lambda i, j: (j, 0)),
        ],
        out_specs=[
            pl.BlockSpec((BQ, BK), lambda i, j: (i, j)),
            pl.BlockSpec((CPB, BQ), lambda i, j: (j, i)),
        ],
        out_shape=[
            jax.ShapeDtypeStruct((Q, K_PAD), jnp.float32),
            jax.ShapeDtypeStruct((N_CHUNK, Q), jnp.float32),
        ],
        compiler_params=pltpu.CompilerParams(
            dimension_semantics=("arbitrary", "arbitrary"),
        ),
    )(qn, en_norm)


def _thr_body(cm_ref, t_ref, mx_ref):
    x = cm_ref[...]
    mx_ref[...] = jnp.max(x, axis=0)[None, :]

    def body(_, xc):
        m = jnp.max(xc, axis=0)
        return jnp.where(xc == m[None, :], -jnp.inf, xc)

    x = lax.fori_loop(0, TOPK - 1, body, x)
    t_ref[...] = jnp.max(x, axis=0)[None, :]


def _thr_call(cmT):
    blk = 512
    return pl.pallas_call(
        _thr_body,
        grid=(Q // blk,),
        in_specs=[pl.BlockSpec((N_CHUNK, blk), lambda i: (0, i))],
        out_specs=[
            pl.BlockSpec((1, blk), lambda i: (0, i)),
            pl.BlockSpec((1, blk), lambda i: (0, i)),
        ],
        out_shape=[
            jax.ShapeDtypeStruct((1, Q), jnp.float32),
            jax.ShapeDtypeStruct((1, Q), jnp.float32),
        ],
    )(cmT)


def _sc_body(sims_hbm, cm_hbm, t_hbm, mx_hbm, es_hbm, out_hbm,
             cmbuf, tbuf, mxbuf, gidx2, chunkbuf, candv, candi,
             wbuf, ibuf, esbuf, accbuf, sem):
    wid = lax.axis_index("s") * 2 + lax.axis_index("c")
    base = wid * RPW
    iota = lax.iota(jnp.int32, 16)
    neg = jnp.full((16,), -1e30, jnp.float32)

    def _lane(v16, lane):
        z = jnp.zeros((16,), v16.dtype)
        return jnp.sum(jnp.where(iota == lane, v16, z))

    def _cnt(m):
        return jnp.sum(jnp.where(m, 1, 0))

    for row in range(GR):
        for sl in range(NG // 16):
            gidx2[row, pl.ds(sl * 16, 16)] = iota + sl * 16
    for sl in range(NG // 16):
        ibuf[pl.ds(sl * 16, 16)] = iota + sl * 16

    pltpu.sync_copy(cm_hbm.at[:, pl.ds(base, RPW)], cmbuf)
    pltpu.sync_copy(t_hbm.at[pl.ds(base, RPW)], tbuf)
    pltpu.sync_copy(mx_hbm.at[pl.ds(base, RPW)], mxbuf)

    def group_body(g, _g):
        rb = g * GR
        qb = base + rb
        t_vec = tbuf[pl.ds(rb, 16)]
        mx_vec = mxbuf[pl.ds(rb, 16)]

        def chunk_body(c, nqv):
            cmrow = cmbuf[c, pl.ds(rb, 16)]
            m = jnp.logical_and(cmrow >= t_vec, nqv < NG)
            plsc.store_scatter(gidx2, [iota, nqv],
                               (qb + iota) * N_CHUNK + c, mask=m)
            return nqv + jnp.where(m, 1, 0)

        nqv = lax.fori_loop(0, N_CHUNK, chunk_body,
                            jnp.zeros((16,), jnp.int32))

        def row_body(rl, _r):
            q = qb + rl
            nq = _lane(nqv, rl)
            t_sp = jnp.full((16,), _lane(t_vec, rl), jnp.float32)
            mx_s = _lane(mx_vec, rl)
            mx_sp = jnp.full((16,), mx_s, jnp.float32)

            pltpu.async_copy(sims_hbm.at[gidx2.at[rl]], chunkbuf, sem).wait()

            for sl in range(NCBUF // 16):
                candv[pl.ds(sl * 16, 16)] = neg

            def cand_body(j, nc):
                g_win = gidx2[rl, pl.ds((j // 16) * 16, 16)]
                cid_sp = jnp.full((16,), _lane(g_win, j % 16) - q * N_CHUNK,
                                  jnp.int32)
                for k in range(8):
                    v = chunkbuf[j, pl.ds(k * 16, 16)]
                    m = v >= t_sp
                    cnt = _cnt(m)

                    @pl.when(jnp.logical_and(nc <= NCAND - 16, cnt > 0))
                    def _():
                        plsc.store_compressed(candv.at[pl.ds(nc, 16)], v,
                                              mask=m)
                        plsc.store_compressed(candi.at[pl.ds(nc, 16)],
                                              cid_sp * 128 + iota + k * 16,
                                              mask=m)
                    nc = nc + jnp.minimum(cnt, NCAND - nc)
                return nc

            lax.fori_loop(0, nq, cand_body, 0)

            cv = tuple(candv[pl.ds(k * 16, 16)] for k in range(NCAND // 16))
            lo0 = _lane(t_vec, rl)
            hi0 = mx_s

            def bis_body(_, lohi):
                lo, hi = lohi
                mid = 0.5 * (lo + hi)
                mid_sp = jnp.full((16,), mid, jnp.float32)
                acc = jnp.zeros((16,), jnp.int32)
                for k in range(NCAND // 16):
                    acc = acc + jnp.where(cv[k] >= mid_sp, 1, 0)
                cnt = jnp.sum(acc)
                ge = cnt >= TOPK
                return (jnp.where(ge, mid, lo), jnp.where(ge, hi, mid))

            lo, _ = lax.fori_loop(0, 30, bis_body, (lo0, hi0))
            tau_sp = jnp.full((16,), lo, jnp.float32)

            nsel = 0
            zv = jnp.zeros((16,), jnp.float32)
            for k in range(NCAND // 16):
                m = cv[k] >= tau_sp
                e = jnp.exp((cv[k] - mx_sp) / TEMP)
                e = jnp.where(m, e, 0.0)
                zv = zv + e
                cnt = _cnt(m)

                @pl.when(jnp.logical_and(nsel <= NG - 16, cnt > 0))
                def _():
                    plsc.store_compressed(wbuf.at[pl.ds(nsel, 16)], e,
                                          mask=m)
                    iv = candi[pl.ds(k * 16, 16)]
                    plsc.store_compressed(ibuf.at[pl.ds(nsel, 16)], iv,
                                          mask=m)
                nsel = nsel + jnp.minimum(cnt, NG - nsel)
            z = jnp.sum(zv)

            pltpu.async_copy(es_hbm.at[ibuf], esbuf, sem).wait()

            def comb_body(j, acc):
                w_win = wbuf[pl.ds((j // 16) * 16, 16)]
                w_sp = jnp.full((16,), _lane(w_win, j % 16), jnp.float32)
                return tuple(acc[k] + w_sp * esbuf[j, pl.ds(k * 16, 16)]
                             for k in range(8))

            acc0 = tuple(jnp.zeros((16,), jnp.float32) for _ in range(8))
            acc = lax.fori_loop(0, nsel, comb_body, acc0)
            z_sp = jnp.full((16,), z, jnp.float32)
            for k in range(8):
                accbuf[rl, pl.ds(k * 16, 16)] = acc[k] / z_sp
            return 0

        lax.fori_loop(0, GR, row_body, 0)
        pltpu.sync_copy(accbuf, out_hbm.at[pl.ds(qb, GR)])
        return 0

    lax.fori_loop(0, RPW // GR, group_body, 0)


def _sc_call(sims2, cmT, t, mx, es_db):
    mesh = plsc.VectorSubcoreMesh(core_axis_name="c", subcore_axis_name="s")
    f = pl.kernel(
        _sc_body, mesh=mesh,
        out_type=jax.ShapeDtypeStruct((Q, D), jnp.float32),
        scratch_types=[
            pltpu.VMEM((N_CHUNK, RPW), jnp.float32),
            pltpu.VMEM((RPW,), jnp.float32),
            pltpu.VMEM((RPW,), jnp.float32),
            pltpu.VMEM((GR, NG), jnp.int32),
            pltpu.VMEM((NG, D), jnp.float32),
            pltpu.VMEM((NCBUF,), jnp.float32),
            pltpu.VMEM((NCBUF,), jnp.int32),
            pltpu.VMEM((NG,), jnp.float32),
            pltpu.VMEM((NG,), jnp.int32),
            pltpu.VMEM((NG, D), jnp.float32),
            pltpu.VMEM((GR, D), jnp.float32),
            pltpu.SemaphoreType.DMA,
        ],
        compiler_params=pltpu.CompilerParams(needs_layout_passes=False),
    )
    return f(sims2, cmT, t, mx, es_db)


def kernel(query, en_db, es_db):
    en_pad = jnp.pad(en_db, ((0, K_PAD - K_DB), (0, 0)))
    en_norm = _l2norm_rows(en_pad, 2048)
    qn = _l2norm_rows(query, 2048)
    sims, cmT = _sims_call(qn, en_norm)
    t2, mx2 = _thr_call(cmT)
    sims2 = sims.reshape(Q * N_CHUNK, D)
    return _sc_call(sims2, cmT, t2.reshape(Q), mx2.reshape(Q), es_db)


# speedup vs baseline: 19.3381x; 1.1552x over previous
import jax
import jax.numpy as jnp
from jax import lax
from jax.experimental import pallas as pl
from jax.experimental.pallas import tpu as pltpu
from jax.experimental.pallas import tpu_sc as plsc

TOPK = 70
TEMP = 0.05
Q = 4096
K_DB = 100000
D = 128
K_PAD = 100352
N_CHUNK = K_PAD // 128

BQ = 512
BK = 2048
CPB = BK // 128

NW = 32
RPW = Q // NW
GR = 16
NG = 96
NCAND = 128
NCBUF = NCAND + 16


def _norm_body(x_ref, o_ref):
    x = x_ref[...]
    n = jnp.sqrt(jnp.sum(x * x, axis=1, keepdims=True))
    o_ref[...] = x / jnp.maximum(n, 1e-12)


def _l2norm_rows(x, blk):
    r = x.shape[0]
    return pl.pallas_call(
        _norm_body,
        grid=(r // blk,),
        in_specs=[pl.BlockSpec((blk, D), lambda i: (i, 0))],
        out_specs=pl.BlockSpec((blk, D), lambda i: (i, 0)),
        out_shape=jax.ShapeDtypeStruct((r, D), jnp.float32),
    )(x)


def _sims_body(q_ref, e_ref, sims_ref, cm_ref):
    j = pl.program_id(1)
    s = lax.dot_general(
        q_ref[...], e_ref[...], (((1,), (1,)), ((), ())),
        preferred_element_type=jnp.float32,
        precision=lax.Precision.DEFAULT,
    )
    col = lax.broadcasted_iota(jnp.int32, (BQ, BK), 1) + j * BK
    s = jnp.where(col < K_DB, s, -1e30)
    sims_ref[...] = s
    cm = jnp.stack(
        [jnp.max(s[:, c * 128:(c + 1) * 128], axis=1) for c in range(CPB)],
        axis=0)
    cm_ref[...] = cm


def _sims_call(qn, en_norm):
    return pl.pallas_call(
        _sims_body,
        grid=(Q // BQ, K_PAD // BK),
        in_specs=[
            pl.BlockSpec((BQ, D), lambda i, j: (i, 0)),
            pl.BlockSpec((BK, D), lambda i, j: (j, 0)),
        ],
        out_specs=[
            pl.BlockSpec((BQ, BK), lambda i, j: (i, j)),
            pl.BlockSpec((CPB, BQ), lambda i, j: (j, i)),
        ],
        out_shape=[
            jax.ShapeDtypeStruct((Q, K_PAD), jnp.float32),
            jax.ShapeDtypeStruct((N_CHUNK, Q), jnp.float32),
        ],
        compiler_params=pltpu.CompilerParams(
            dimension_semantics=("arbitrary", "arbitrary"),
        ),
    )(qn, en_norm)


def _thr_body(cm_ref, t_ref, mx_ref):
    x = cm_ref[...]
    mx_ref[...] = jnp.max(x, axis=0)[None, :]

    def body(_, xc):
        m = jnp.max(xc, axis=0)
        return jnp.where(xc == m[None, :], -jnp.inf, xc)

    x = lax.fori_loop(0, TOPK - 1, body, x)
    t_ref[...] = jnp.max(x, axis=0)[None, :]


def _thr_call(cmT):
    blk = 512
    return pl.pallas_call(
        _thr_body,
        grid=(Q // blk,),
        in_specs=[pl.BlockSpec((N_CHUNK, blk), lambda i: (0, i))],
        out_specs=[
            pl.BlockSpec((1, blk), lambda i: (0, i)),
            pl.BlockSpec((1, blk), lambda i: (0, i)),
        ],
        out_shape=[
            jax.ShapeDtypeStruct((1, Q), jnp.float32),
            jax.ShapeDtypeStruct((1, Q), jnp.float32),
        ],
    )(cmT)


def _sc_body(sims_hbm, cm_hbm, t_hbm, mx_hbm, es_hbm, out_hbm,
             cmbuf, tbuf, mxbuf, nqbuf, gidx2, chunkbuf, candv, candi,
             wbuf, ibuf, esbuf, accbuf, sem):
    wid = lax.axis_index("s") * 2 + lax.axis_index("c")
    base = wid * RPW
    iota = lax.iota(jnp.int32, 16)
    ones = jnp.full((16,), 1, jnp.int32)
    zeros = jnp.zeros((16,), jnp.int32)

    def _sp(ref, j):
        return plsc.load_gather(ref, [jnp.full((16,), j, jnp.int32)])

    for row in range(GR):
        for sl in range(NG // 16):
            gidx2[row, pl.ds(sl * 16, 16)] = iota + sl * 16
    for sl in range(NG // 16):
        ibuf[pl.ds(sl * 16, 16)] = iota + sl * 16

    pltpu.sync_copy(cm_hbm.at[:, pl.ds(base, RPW)], cmbuf)
    pltpu.sync_copy(t_hbm.at[pl.ds(base, RPW)], tbuf)
    pltpu.sync_copy(mx_hbm.at[pl.ds(base, RPW)], mxbuf)

    def group_body(g, _g):
        rb = g * GR
        qb = base + rb
        t_vec = tbuf[pl.ds(rb, 16)]

        def chunk_body(c, nqv):
            cmrow = cmbuf[c, pl.ds(rb, 16)]
            m = jnp.logical_and(cmrow >= t_vec, nqv < NG)
            plsc.store_scatter(gidx2, [iota, nqv],
                               (qb + iota) * N_CHUNK + c, mask=m)
            return nqv + jnp.where(m, 1, 0)

        nqv = lax.fori_loop(0, N_CHUNK, chunk_body, zeros)
        nqbuf[...] = nqv

        def row_body(rl, _r):
            q = qb + rl
            nq = jnp.max(_sp(nqbuf, rl))
            t_sp = _sp(tbuf, rb + rl)
            mx_sp = _sp(mxbuf, rb + rl)

            pltpu.async_copy(sims_hbm.at[gidx2.at[rl]], chunkbuf, sem).wait()

            neg = jnp.full((16,), -1e30, jnp.float32)
            for sl in range(NCBUF // 16):
                candv[pl.ds(sl * 16, 16)] = neg

            def cand_body(j, nc_v):
                gj = plsc.load_gather(
                    gidx2, [jnp.full((16,), rl, jnp.int32),
                            jnp.full((16,), j, jnp.int32)])
                cid_sp = (gj - q * N_CHUNK) * 128
                off = nc_v
                for k in range(8):
                    v = chunkbuf[j, pl.ds(k * 16, 16)]
                    m = v >= t_sp
                    pc = plsc.all_reduce_population_count(m)
                    pos = off + plsc.cumsum(jnp.where(m, ones, zeros)) - 1
                    okm = jnp.logical_and(m, pos < NCAND)
                    plsc.store_scatter(candv, [pos], v, mask=okm)
                    plsc.store_scatter(candi, [pos],
                                       cid_sp + iota + k * 16, mask=okm)
                    off = off + pc
                return off

            lax.fori_loop(0, nq, cand_body, zeros)

            cv = tuple(candv[pl.ds(k * 16, 16)] for k in range(NCAND // 16))

            def bis_body(_, lohi):
                lo, hi = lohi
                mid = 0.5 * (lo + hi)
                acc = zeros
                for k in range(NCAND // 16):
                    acc = acc + plsc.all_reduce_population_count(cv[k] >= mid)
                ge = acc >= TOPK
                return (jnp.where(ge, mid, lo), jnp.where(ge, hi, mid))

            tau_sp, _ = lax.fori_loop(0, 30, bis_body, (t_sp, mx_sp))

            ns_v = zeros
            zv = jnp.zeros((16,), jnp.float32)
            for k in range(NCAND // 16):
                m = cv[k] >= tau_sp
                e = jnp.exp((cv[k] - mx_sp) / TEMP)
                e = jnp.where(m, e, 0.0)
                zv = zv + e
                pc = plsc.all_reduce_population_count(m)
                pos = ns_v + plsc.cumsum(jnp.where(m, ones, zeros)) - 1
                okm = jnp.logical_and(m, pos < NG)
                plsc.store_scatter(wbuf, [pos], e, mask=okm)
                iv = candi[pl.ds(k * 16, 16)]
                plsc.store_scatter(ibuf, [pos], iv, mask=okm)
                ns_v = ns_v + pc
            z = jnp.sum(zv)
            nsel = jnp.max(jnp.minimum(ns_v, NG))

            pltpu.async_copy(es_hbm.at[ibuf], esbuf, sem).wait()

            def comb_body(j, acc):
                w_sp = _sp(wbuf, j)
                return tuple(acc[k] + w_sp * esbuf[j, pl.ds(k * 16, 16)]
                             for k in range(8))

            acc0 = tuple(jnp.zeros((16,), jnp.float32) for _ in range(8))
            acc = lax.fori_loop(0, nsel, comb_body, acc0)
            z_sp = jnp.full((16,), z, jnp.float32)
            for k in range(8):
                accbuf[rl, pl.ds(k * 16, 16)] = acc[k] / z_sp
            return 0

        lax.fori_loop(0, GR, row_body, 0)
        pltpu.sync_copy(accbuf, out_hbm.at[pl.ds(qb, GR)])
        return 0

    lax.fori_loop(0, RPW // GR, group_body, 0)


def _sc_call(sims2, cmT, t, mx, es_db):
    mesh = plsc.VectorSubcoreMesh(core_axis_name="c", subcore_axis_name="s")
    f = pl.kernel(
        _sc_body, mesh=mesh,
        out_type=jax.ShapeDtypeStruct((Q, D), jnp.float32),
        scratch_types=[
            pltpu.VMEM((N_CHUNK, RPW), jnp.float32),
            pltpu.VMEM((RPW,), jnp.float32),
            pltpu.VMEM((RPW,), jnp.float32),
            pltpu.VMEM((16,), jnp.int32),
            pltpu.VMEM((GR, NG), jnp.int32),
            pltpu.VMEM((NG, D), jnp.float32),
            pltpu.VMEM((NCBUF,), jnp.float32),
            pltpu.VMEM((NCBUF,), jnp.int32),
            pltpu.VMEM((NG,), jnp.float32),
            pltpu.VMEM((NG,), jnp.int32),
            pltpu.VMEM((NG, D), jnp.float32),
            pltpu.VMEM((GR, D), jnp.float32),
            pltpu.SemaphoreType.DMA,
        ],
        compiler_params=pltpu.CompilerParams(needs_layout_passes=False),
    )
    return f(sims2, cmT, t, mx, es_db)


def kernel(query, en_db, es_db):
    en_pad = jnp.pad(en_db, ((0, K_PAD - K_DB), (0, 0)))
    en_norm = _l2norm_rows(en_pad, 2048)
    qn = _l2norm_rows(query, 2048)
    sims, cmT = _sims_call(qn, en_norm)
    t2, mx2 = _thr_call(cmT)
    sims2 = sims.reshape(Q * N_CHUNK, D)
    return _sc_call(sims2, cmT, t2.reshape(Q), mx2.reshape(Q), es_db)


# speedup vs baseline: 20.4083x; 1.0553x over previous
import jax
import jax.numpy as jnp
from jax import lax
from jax.experimental import pallas as pl
from jax.experimental.pallas import tpu as pltpu
from jax.experimental.pallas import tpu_sc as plsc

TOPK = 70
TEMP = 0.05
Q = 4096
K_DB = 100000
D = 128
K_PAD = 100352
N_CHUNK = K_PAD // 128

BQ = 512
BK = 2048
CPB = BK // 128

NW = 32
RPW = Q // NW
GR = 16
NG = 96
NCAND = 128
NCBUF = NCAND + 16


def _norm_body(x_ref, o_ref):
    x = x_ref[...]
    n = jnp.sqrt(jnp.sum(x * x, axis=1, keepdims=True))
    o_ref[...] = x / jnp.maximum(n, 1e-12)


def _l2norm_rows(x, blk):
    r = x.shape[0]
    return pl.pallas_call(
        _norm_body,
        grid=(r // blk,),
        in_specs=[pl.BlockSpec((blk, D), lambda i: (i, 0))],
        out_specs=pl.BlockSpec((blk, D), lambda i: (i, 0)),
        out_shape=jax.ShapeDtypeStruct((r, D), jnp.float32),
    )(x)


def _sims_body(q_ref, e_ref, sims_ref, cm_ref, t_ref, mx_ref, cma_ref):
    j = pl.program_id(1)
    s = lax.dot_general(
        q_ref[...], e_ref[...], (((1,), (1,)), ((), ())),
        preferred_element_type=jnp.float32,
        precision=lax.Precision.DEFAULT,
    )
    col = lax.broadcasted_iota(jnp.int32, (BQ, BK), 1) + j * BK
    s = jnp.where(col < K_DB, s, -1e30)
    sims_ref[...] = s
    cm = jnp.stack(
        [jnp.max(s[:, c * 128:(c + 1) * 128], axis=1) for c in range(CPB)],
        axis=0)
    cm_ref[...] = cm
    cma_ref[pl.ds(j * CPB, CPB), :] = cm

    @pl.when(j == (K_PAD // BK) - 1)
    def _():
        cma = cma_ref[...]
        mx = jnp.max(cma, axis=0)

        def bb(_, lohi):
            lo, hi = lohi
            mid = 0.5 * (lo + hi)
            cnt = jnp.sum(jnp.where(cma >= mid[None, :], 1, 0), axis=0)
            ge = cnt >= TOPK
            return (jnp.where(ge, mid, lo), jnp.where(ge, hi, mid))

        lo, _ = lax.fori_loop(
            0, 24, bb, (jnp.full((BQ,), -1.0, jnp.float32), mx))
        t_ref[...] = lo[None, :]
        mx_ref[...] = mx[None, :]


def _sims_call(qn, en_norm):
    return pl.pallas_call(
        _sims_body,
        grid=(Q // BQ, K_PAD // BK),
        in_specs=[
            pl.BlockSpec((BQ, D), lambda i, j: (i, 0)),
            pl.BlockSpec((BK, D), lambda i, j: (j, 0)),
        ],
        out_specs=[
            pl.BlockSpec((BQ, BK), lambda i, j: (i, j)),
            pl.BlockSpec((CPB, BQ), lambda i, j: (j, i)),
            pl.BlockSpec((1, BQ), lambda i, j: (0, i)),
            pl.BlockSpec((1, BQ), lambda i, j: (0, i)),
        ],
        out_shape=[
            jax.ShapeDtypeStruct((Q, K_PAD), jnp.float32),
            jax.ShapeDtypeStruct((N_CHUNK, Q), jnp.float32),
            jax.ShapeDtypeStruct((1, Q), jnp.float32),
            jax.ShapeDtypeStruct((1, Q), jnp.float32),
        ],
        scratch_shapes=[pltpu.VMEM((N_CHUNK, BQ), jnp.float32)],
        compiler_params=pltpu.CompilerParams(
            dimension_semantics=("arbitrary", "arbitrary"),
        ),
    )(qn, en_norm)


def _sc_body(sims_hbm, cm_hbm, t_hbm, mx_hbm, es_hbm, out_hbm,
             cmbuf, tbuf, mxbuf, nqbuf, gidx2, chunkbuf, candv, candi,
             wbuf, ibuf, esbuf, accbuf, sem):
    wid = lax.axis_index("s") * 2 + lax.axis_index("c")
    base = wid * RPW
    iota = lax.iota(jnp.int32, 16)
    ones = jnp.full((16,), 1, jnp.int32)
    zeros = jnp.zeros((16,), jnp.int32)

    def _sp(ref, j):
        return plsc.load_gather(ref, [jnp.full((16,), j, jnp.int32)])

    for row in range(GR):
        for sl in range(NG // 16):
            gidx2[row, pl.ds(sl * 16, 16)] = iota + sl * 16
    for sl in range(NG // 16):
        ibuf[pl.ds(sl * 16, 16)] = iota + sl * 16

    pltpu.sync_copy(cm_hbm.at[:, pl.ds(base, RPW)], cmbuf)
    pltpu.sync_copy(t_hbm.at[pl.ds(base, RPW)], tbuf)
    pltpu.sync_copy(mx_hbm.at[pl.ds(base, RPW)], mxbuf)

    def group_body(g, _g):
        rb = g * GR
        qb = base + rb
        t_vec = tbuf[pl.ds(rb, 16)]

        def chunk_body(c, nqv):
            cmrow = cmbuf[c, pl.ds(rb, 16)]
            m = jnp.logical_and(cmrow >= t_vec, nqv < NG)
            plsc.store_scatter(gidx2, [iota, nqv],
                               (qb + iota) * N_CHUNK + c, mask=m)
            return nqv + jnp.where(m, 1, 0)

        nqv = lax.fori_loop(0, N_CHUNK, chunk_body, zeros)
        nqbuf[...] = nqv

        def row_body(rl, _r):
            q = qb + rl
            nq = jnp.max(_sp(nqbuf, rl))
            t_sp = _sp(tbuf, rb + rl)
            mx_sp = _sp(mxbuf, rb + rl)

            pltpu.async_copy(sims_hbm.at[gidx2.at[rl]], chunkbuf, sem).wait()

            neg = jnp.full((16,), -1e30, jnp.float32)
            for sl in range(NCBUF // 16):
                candv[pl.ds(sl * 16, 16)] = neg

            def cand_body(j, nc_v):
                gj = plsc.load_gather(
                    gidx2, [jnp.full((16,), rl, jnp.int32),
                            jnp.full((16,), j, jnp.int32)])
                cid_sp = (gj - q * N_CHUNK) * 128
                off = nc_v
                for k in range(8):
                    v = chunkbuf[j, pl.ds(k * 16, 16)]
                    m = v >= t_sp
                    pc = plsc.all_reduce_population_count(m)
                    pos = off + plsc.cumsum(jnp.where(m, ones, zeros)) - 1
                    okm = jnp.logical_and(m, pos < NCAND)
                    plsc.store_scatter(candv, [pos], v, mask=okm)
                    plsc.store_scatter(candi, [pos],
                                       cid_sp + iota + k * 16, mask=okm)
                    off = off + pc
                return off

            lax.fori_loop(0, nq, cand_body, zeros)

            cv = tuple(candv[pl.ds(k * 16, 16)] for k in range(NCAND // 16))

            def bis_body(_, lohi):
                lo, hi = lohi
                mid = 0.5 * (lo + hi)
                acc = zeros
                for k in range(NCAND // 16):
                    acc = acc + plsc.all_reduce_population_count(cv[k] >= mid)
                ge = acc >= TOPK
                return (jnp.where(ge, mid, lo), jnp.where(ge, hi, mid))

            tau_sp, _ = lax.fori_loop(0, 30, bis_body, (t_sp, mx_sp))

            ns_v = zeros
            zv = jnp.zeros((16,), jnp.float32)
            for k in range(NCAND // 16):
                m = cv[k] >= tau_sp
                e = jnp.exp((cv[k] - mx_sp) / TEMP)
                e = jnp.where(m, e, 0.0)
                zv = zv + e
                pc = plsc.all_reduce_population_count(m)
                pos = ns_v + plsc.cumsum(jnp.where(m, ones, zeros)) - 1
                okm = jnp.logical_and(m, pos < NG)
                plsc.store_scatter(wbuf, [pos], e, mask=okm)
                iv = candi[pl.ds(k * 16, 16)]
                plsc.store_scatter(ibuf, [pos], iv, mask=okm)
                ns_v = ns_v + pc
            z = jnp.sum(zv)
            nsel = jnp.max(jnp.minimum(ns_v, NG))

            pltpu.async_copy(es_hbm.at[ibuf], esbuf, sem).wait()

            def comb_body(j, acc):
                w_sp = _sp(wbuf, j)
                return tuple(acc[k] + w_sp * esbuf[j, pl.ds(k * 16, 16)]
                             for k in range(8))

            acc0 = tuple(jnp.zeros((16,), jnp.float32) for _ in range(8))
            acc = lax.fori_loop(0, nsel, comb_body, acc0)
            z_sp = jnp.full((16,), z, jnp.float32)
            for k in range(8):
                accbuf[rl, pl.ds(k * 16, 16)] = acc[k] / z_sp
            return 0

        lax.fori_loop(0, GR, row_body, 0)
        pltpu.sync_copy(accbuf, out_hbm.at[pl.ds(qb, GR)])
        return 0

    lax.fori_loop(0, RPW // GR, group_body, 0)


def _sc_call(sims2, cmT, t, mx, es_db):
    mesh = plsc.VectorSubcoreMesh(core_axis_name="c", subcore_axis_name="s")
    f = pl.kernel(
        _sc_body, mesh=mesh,
        out_type=jax.ShapeDtypeStruct((Q, D), jnp.float32),
        scratch_types=[
            pltpu.VMEM((N_CHUNK, RPW), jnp.float32),
            pltpu.VMEM((RPW,), jnp.float32),
            pltpu.VMEM((RPW,), jnp.float32),
            pltpu.VMEM((16,), jnp.int32),
            pltpu.VMEM((GR, NG), jnp.int32),
            pltpu.VMEM((NG, D), jnp.float32),
            pltpu.VMEM((NCBUF,), jnp.float32),
            pltpu.VMEM((NCBUF,), jnp.int32),
            pltpu.VMEM((NG,), jnp.float32),
            pltpu.VMEM((NG,), jnp.int32),
            pltpu.VMEM((NG, D), jnp.float32),
            pltpu.VMEM((GR, D), jnp.float32),
            pltpu.SemaphoreType.DMA,
        ],
        compiler_params=pltpu.CompilerParams(needs_layout_passes=False),
    )
    return f(sims2, cmT, t, mx, es_db)


def kernel(query, en_db, es_db):
    en_pad = jnp.pad(en_db, ((0, K_PAD - K_DB), (0, 0)))
    en_norm = _l2norm_rows(en_pad, 2048)
    qn = _l2norm_rows(query, 2048)
    sims, cmT, t2, mx2 = _sims_call(qn, en_norm)
    sims2 = sims.reshape(Q * N_CHUNK, D)
    return _sc_call(sims2, cmT, t2.reshape(Q), mx2.reshape(Q), es_db)


# speedup vs baseline: 26.1553x; 1.2816x over previous
import jax
import jax.numpy as jnp
from jax import lax
from jax.experimental import pallas as pl
from jax.experimental.pallas import tpu as pltpu
from jax.experimental.pallas import tpu_sc as plsc

TOPK = 70
TEMP = 0.05
Q = 4096
K_DB = 100000
D = 128
K_PAD = 100352
N_CHUNK = K_PAD // 128

BQ = 512
BK = 2048
CPB = BK // 128

NW = 32
SLAB = 1024
NSLAB = Q // SLAB
RPW = SLAB // NW
GR = 16
NG = 96
NCAND = 128
NCBUF = NCAND + 16


def _norm_body(x_ref, o_ref):
    x = x_ref[...]
    n = jnp.sqrt(jnp.sum(x * x, axis=1, keepdims=True))
    o_ref[...] = x / jnp.maximum(n, 1e-12)


def _l2norm_rows(x, blk):
    r = x.shape[0]
    return pl.pallas_call(
        _norm_body,
        grid=(r // blk,),
        in_specs=[pl.BlockSpec((blk, D), lambda i: (i, 0))],
        out_specs=pl.BlockSpec((blk, D), lambda i: (i, 0)),
        out_shape=jax.ShapeDtypeStruct((r, D), jnp.float32),
    )(x)


def _sims_body(q_ref, e_ref, sims_ref, cm_ref, t_ref, mx_ref, cma_ref):
    j = pl.program_id(1)
    s = lax.dot_general(
        q_ref[...], e_ref[...], (((1,), (1,)), ((), ())),
        preferred_element_type=jnp.float32,
        precision=lax.Precision.DEFAULT,
    )
    col = lax.broadcasted_iota(jnp.int32, (BQ, BK), 1) + j * BK
    s = jnp.where(col < K_DB, s, -1e30)
    sims_ref[...] = s
    cm = jnp.stack(
        [jnp.max(s[:, c * 128:(c + 1) * 128], axis=1) for c in range(CPB)],
        axis=0)
    cm_ref[...] = cm
    cma_ref[pl.ds(j * CPB, CPB), :] = cm

    @pl.when(j == (K_PAD // BK) - 1)
    def _():
        cma = cma_ref[...]
        mx = jnp.max(cma, axis=0)

        def bb(_, lohi):
            lo, hi = lohi
            mid = 0.5 * (lo + hi)
            cnt = jnp.sum(jnp.where(cma >= mid[None, :], 1, 0), axis=0)
            ge = cnt >= TOPK
            return (jnp.where(ge, mid, lo), jnp.where(ge, hi, mid))

        lo, _ = lax.fori_loop(
            0, 24, bb, (jnp.full((BQ,), -1.0, jnp.float32), mx))
        t_ref[...] = lo[None, :]
        mx_ref[...] = mx[None, :]


def _sims_call(qn_slab, en_norm):
    return pl.pallas_call(
        _sims_body,
        grid=(SLAB // BQ, K_PAD // BK),
        in_specs=[
            pl.BlockSpec((BQ, D), lambda i, j: (i, 0)),
            pl.BlockSpec((BK, D), lambda i, j: (j, 0)),
        ],
        out_specs=[
            pl.BlockSpec((BQ, BK), lambda i, j: (i, j)),
            pl.BlockSpec((CPB, BQ), lambda i, j: (j, i)),
            pl.BlockSpec((1, BQ), lambda i, j: (0, i)),
            pl.BlockSpec((1, BQ), lambda i, j: (0, i)),
        ],
        out_shape=[
            jax.ShapeDtypeStruct((SLAB, K_PAD), jnp.float32),
            jax.ShapeDtypeStruct((N_CHUNK, SLAB), jnp.float32),
            jax.ShapeDtypeStruct((1, SLAB), jnp.float32),
            jax.ShapeDtypeStruct((1, SLAB), jnp.float32),
        ],
        scratch_shapes=[pltpu.VMEM((N_CHUNK, BQ), jnp.float32)],
        compiler_params=pltpu.CompilerParams(
            dimension_semantics=("arbitrary", "arbitrary"),
        ),
    )(qn_slab, en_norm)


def _sc_body(sims_hbm, cm_hbm, t_hbm, mx_hbm, es_hbm, out_hbm,
             cmbuf, tbuf, mxbuf, nqbuf, gidx2, chunkbuf, candv, candi,
             wbuf, ibuf, esbuf, accbuf, sem):
    wid = lax.axis_index("s") * 2 + lax.axis_index("c")
    cb128 = (wid // 4) * 128
    sub = (wid % 4) * RPW
    w0 = cb128 + sub
    iota = lax.iota(jnp.int32, 16)
    ones = jnp.full((16,), 1, jnp.int32)
    zeros = jnp.zeros((16,), jnp.int32)

    def _sp(ref, j):
        return plsc.load_gather(ref, [jnp.full((16,), j, jnp.int32)])

    for row in range(GR):
        for sl in range(NG // 16):
            gidx2[row, pl.ds(sl * 16, 16)] = iota + sl * 16
    for sl in range(NG // 16):
        ibuf[pl.ds(sl * 16, 16)] = iota + sl * 16

    pltpu.sync_copy(cm_hbm.at[:, pl.ds(cb128, 128)], cmbuf)
    pltpu.sync_copy(t_hbm.at[pl.ds(w0, RPW)], tbuf)
    pltpu.sync_copy(mx_hbm.at[pl.ds(w0, RPW)], mxbuf)

    def group_body(g, _g):
        rb = g * GR
        rbc = sub + g * GR
        qb = w0 + g * GR
        t_vec = tbuf[pl.ds(rb, 16)]

        def chunk_body(c, nqv):
            cmrow = cmbuf[c, pl.ds(rbc, 16)]
            m = jnp.logical_and(cmrow >= t_vec, nqv < NG)
            plsc.store_scatter(gidx2, [iota, nqv],
                               (qb + iota) * N_CHUNK + c, mask=m)
            return nqv + jnp.where(m, 1, 0)

        nqv = lax.fori_loop(0, N_CHUNK, chunk_body, zeros)
        nqbuf[...] = nqv

        def row_body(rl, _r):
            q = qb + rl
            nq = jnp.max(_sp(nqbuf, rl))
            t_sp = _sp(tbuf, rb + rl)
            mx_sp = _sp(mxbuf, rb + rl)

            pltpu.async_copy(sims_hbm.at[gidx2.at[rl]], chunkbuf, sem).wait()

            neg = jnp.full((16,), -1e30, jnp.float32)
            for sl in range(NCBUF // 16):
                candv[pl.ds(sl * 16, 16)] = neg

            def cand_body(j, nc_v):
                gj = plsc.load_gather(
                    gidx2, [jnp.full((16,), rl, jnp.int32),
                            jnp.full((16,), j, jnp.int32)])
                cid_sp = (gj - q * N_CHUNK) * 128
                off = nc_v
                for k in range(8):
                    v = chunkbuf[j, pl.ds(k * 16, 16)]
                    m = v >= t_sp
                    pc = plsc.all_reduce_population_count(m)
                    pos = off + plsc.cumsum(jnp.where(m, ones, zeros)) - 1
                    okm = jnp.logical_and(m, pos < NCAND)
                    plsc.store_scatter(candv, [pos], v, mask=okm)
                    plsc.store_scatter(candi, [pos],
                                       cid_sp + iota + k * 16, mask=okm)
                    off = off + pc
                return off

            lax.fori_loop(0, nq, cand_body, zeros)

            cv = tuple(candv[pl.ds(k * 16, 16)] for k in range(NCAND // 16))

            def bis_body(_, lohi):
                lo, hi = lohi
                mid = 0.5 * (lo + hi)
                acc = zeros
                for k in range(NCAND // 16):
                    acc = acc + plsc.all_reduce_population_count(cv[k] >= mid)
                ge = acc >= TOPK
                return (jnp.where(ge, mid, lo), jnp.where(ge, hi, mid))

            tau_sp, _ = lax.fori_loop(0, 30, bis_body, (t_sp, mx_sp))

            ns_v = zeros
            zv = jnp.zeros((16,), jnp.float32)
            for k in range(NCAND // 16):
                m = cv[k] >= tau_sp
                e = jnp.exp((cv[k] - mx_sp) / TEMP)
                e = jnp.where(m, e, 0.0)
                zv = zv + e
                pc = plsc.all_reduce_population_count(m)
                pos = ns_v + plsc.cumsum(jnp.where(m, ones, zeros)) - 1
                okm = jnp.logical_and(m, pos < NG)
                plsc.store_scatter(wbuf, [pos], e, mask=okm)
                iv = candi[pl.ds(k * 16, 16)]
                plsc.store_scatter(ibuf, [pos], iv, mask=okm)
                ns_v = ns_v + pc
            z = jnp.sum(zv)
            nsel = jnp.max(jnp.minimum(ns_v, NG))

            pltpu.async_copy(es_hbm.at[ibuf], esbuf, sem).wait()

            def comb_body(j, acc):
                w_sp = _sp(wbuf, j)
                return tuple(acc[k] + w_sp * esbuf[j, pl.ds(k * 16, 16)]
                             for k in range(8))

            acc0 = tuple(jnp.zeros((16,), jnp.float32) for _ in range(8))
            acc = lax.fori_loop(0, nsel, comb_body, acc0)
            z_sp = jnp.full((16,), z, jnp.float32)
            for k in range(8):
                accbuf[rl, pl.ds(k * 16, 16)] = acc[k] / z_sp
            return 0

        lax.fori_loop(0, GR, row_body, 0)
        pltpu.sync_copy(accbuf, out_hbm.at[pl.ds(qb, GR)])
        return 0

    lax.fori_loop(0, RPW // GR, group_body, 0)


def _sc_call(sims2, cmT, t, mx, es_db):
    mesh = plsc.VectorSubcoreMesh(core_axis_name="c", subcore_axis_name="s")
    f = pl.kernel(
        _sc_body, mesh=mesh,
        out_type=jax.ShapeDtypeStruct((SLAB, D), jnp.float32),
        scratch_types=[
            pltpu.VMEM((N_CHUNK, 128), jnp.float32),
            pltpu.VMEM((RPW,), jnp.float32),
            pltpu.VMEM((RPW,), jnp.float32),
            pltpu.VMEM((16,), jnp.int32),
            pltpu.VMEM((GR, NG), jnp.int32),
            pltpu.VMEM((NG, D), jnp.float32),
            pltpu.VMEM((NCBUF,), jnp.float32),
            pltpu.VMEM((NCBUF,), jnp.int32),
            pltpu.VMEM((NG,), jnp.float32),
            pltpu.VMEM((NG,), jnp.int32),
            pltpu.VMEM((NG, D), jnp.float32),
            pltpu.VMEM((GR, D), jnp.float32),
            pltpu.SemaphoreType.DMA,
        ],
        compiler_params=pltpu.CompilerParams(needs_layout_passes=False),
    )
    return f(sims2, cmT, t, mx, es_db)


def kernel(query, en_db, es_db):
    en_pad = jnp.pad(en_db, ((0, K_PAD - K_DB), (0, 0)))
    en_norm = _l2norm_rows(en_pad, 2048)
    qn = _l2norm_rows(query, 2048)
    outs = []
    for s in range(NSLAB):
        qs = lax.slice_in_dim(qn, s * SLAB, (s + 1) * SLAB, axis=0)
        sims, cmT, t2, mx2 = _sims_call(qs, en_norm)
        sims2 = sims.reshape(SLAB * N_CHUNK, D)
        outs.append(_sc_call(sims2, cmT, t2.reshape(SLAB),
                             mx2.reshape(SLAB), es_db))
    return jnp.concatenate(outs, axis=0)


# speedup vs baseline: 27.7552x; 1.0612x over previous
import jax
import jax.numpy as jnp
from jax import lax
from jax.experimental import pallas as pl
from jax.experimental.pallas import tpu as pltpu
from jax.experimental.pallas import tpu_sc as plsc

TOPK = 70
TEMP = 0.05
Q = 4096
K_DB = 100000
D = 128
K_PAD = 100352
N_CHUNK = K_PAD // 128

BQ = 512
BK = 2048
CPB = BK // 128

NW = 32
SLAB = 512
NSLAB = Q // SLAB
RPW = SLAB // NW
GR = 16
NG = 96
NCAND = 128
NCBUF = NCAND + 16


def _norm_body(x_ref, o_ref):
    x = x_ref[...]
    n = jnp.sqrt(jnp.sum(x * x, axis=1, keepdims=True))
    o_ref[...] = x / jnp.maximum(n, 1e-12)


def _l2norm_rows(x, blk):
    r = x.shape[0]
    return pl.pallas_call(
        _norm_body,
        grid=(r // blk,),
        in_specs=[pl.BlockSpec((blk, D), lambda i: (i, 0))],
        out_specs=pl.BlockSpec((blk, D), lambda i: (i, 0)),
        out_shape=jax.ShapeDtypeStruct((r, D), jnp.float32),
    )(x)


def _sims_body(q_ref, e_ref, sims_ref, cm_ref, t_ref, mx_ref, cma_ref):
    j = pl.program_id(1)
    s = lax.dot_general(
        q_ref[...], e_ref[...], (((1,), (1,)), ((), ())),
        preferred_element_type=jnp.float32,
        precision=lax.Precision.DEFAULT,
    )
    col = lax.broadcasted_iota(jnp.int32, (BQ, BK), 1) + j * BK
    s = jnp.where(col < K_DB, s, -1e30)
    sims_ref[...] = s
    cm = jnp.stack(
        [jnp.max(s[:, c * 128:(c + 1) * 128], axis=1) for c in range(CPB)],
        axis=0)
    cm_ref[...] = cm
    cma_ref[pl.ds(j * CPB, CPB), :] = cm

    @pl.when(j == (K_PAD // BK) - 1)
    def _():
        cma = cma_ref[...]
        mx = jnp.max(cma, axis=0)

        def bb(_, lohi):
            lo, hi = lohi
            mid = 0.5 * (lo + hi)
            cnt = jnp.sum(jnp.where(cma >= mid[None, :], 1, 0), axis=0)
            ge = cnt >= TOPK
            return (jnp.where(ge, mid, lo), jnp.where(ge, hi, mid))

        lo, _ = lax.fori_loop(
            0, 24, bb, (jnp.full((BQ,), -1.0, jnp.float32), mx))
        t_ref[...] = lo[None, :]
        mx_ref[...] = mx[None, :]


def _sims_call(qn_slab, en_norm):
    return pl.pallas_call(
        _sims_body,
        grid=(SLAB // BQ, K_PAD // BK),
        in_specs=[
            pl.BlockSpec((BQ, D), lambda i, j: (i, 0)),
            pl.BlockSpec((BK, D), lambda i, j: (j, 0)),
        ],
        out_specs=[
            pl.BlockSpec((BQ, BK), lambda i, j: (i, j)),
            pl.BlockSpec((CPB, BQ), lambda i, j: (j, i)),
            pl.BlockSpec((1, BQ), lambda i, j: (0, i)),
            pl.BlockSpec((1, BQ), lambda i, j: (0, i)),
        ],
        out_shape=[
            jax.ShapeDtypeStruct((SLAB, K_PAD), jnp.float32),
            jax.ShapeDtypeStruct((N_CHUNK, SLAB), jnp.float32),
            jax.ShapeDtypeStruct((1, SLAB), jnp.float32),
            jax.ShapeDtypeStruct((1, SLAB), jnp.float32),
        ],
        scratch_shapes=[pltpu.VMEM((N_CHUNK, BQ), jnp.float32)],
        compiler_params=pltpu.CompilerParams(
            dimension_semantics=("arbitrary", "arbitrary"),
        ),
    )(qn_slab, en_norm)


def _sc_body(sims_hbm, cm_hbm, t_hbm, mx_hbm, es_hbm, out_hbm,
             cmbuf, tbuf, mxbuf, nqbuf, gidx2, chunkbuf, candv, candi,
             wbuf, ibuf, esbuf, accbuf, sem):
    wid = lax.axis_index("s") * 2 + lax.axis_index("c")
    per_blk = 128 // RPW
    cb128 = (wid // per_blk) * 128
    sub = (wid % per_blk) * RPW
    w0 = cb128 + sub
    iota = lax.iota(jnp.int32, 16)
    ones = jnp.full((16,), 1, jnp.int32)
    zeros = jnp.zeros((16,), jnp.int32)

    def _sp(ref, j):
        return plsc.load_gather(ref, [jnp.full((16,), j, jnp.int32)])

    for row in range(GR):
        for sl in range(NG // 16):
            gidx2[row, pl.ds(sl * 16, 16)] = iota + sl * 16
    for sl in range(NG // 16):
        ibuf[pl.ds(sl * 16, 16)] = iota + sl * 16

    pltpu.sync_copy(cm_hbm.at[:, pl.ds(cb128, 128)], cmbuf)
    pltpu.sync_copy(t_hbm.at[pl.ds(w0, RPW)], tbuf)
    pltpu.sync_copy(mx_hbm.at[pl.ds(w0, RPW)], mxbuf)

    def group_body(g, _g):
        rb = g * GR
        rbc = sub + g * GR
        qb = w0 + g * GR
        t_vec = tbuf[pl.ds(rb, 16)]

        def chunk_body(c, nqv):
            cmrow = cmbuf[c, pl.ds(rbc, 16)]
            m = jnp.logical_and(cmrow >= t_vec, nqv < NG)
            plsc.store_scatter(gidx2, [iota, nqv],
                               (qb + iota) * N_CHUNK + c, mask=m)
            return nqv + jnp.where(m, 1, 0)

        nqv = lax.fori_loop(0, N_CHUNK, chunk_body, zeros)
        nqbuf[...] = nqv

        def row_body(rl, _r):
            q = qb + rl
            nq = jnp.max(_sp(nqbuf, rl))
            t_sp = _sp(tbuf, rb + rl)
            mx_sp = _sp(mxbuf, rb + rl)

            pltpu.async_copy(sims_hbm.at[gidx2.at[rl]], chunkbuf, sem).wait()

            neg = jnp.full((16,), -1e30, jnp.float32)
            for sl in range(NCBUF // 16):
                candv[pl.ds(sl * 16, 16)] = neg

            def cand_body(j, nc_v):
                gj = plsc.load_gather(
                    gidx2, [jnp.full((16,), rl, jnp.int32),
                            jnp.full((16,), j, jnp.int32)])
                cid_sp = (gj - q * N_CHUNK) * 128
                off = nc_v
                for k in range(8):
                    v = chunkbuf[j, pl.ds(k * 16, 16)]
                    m = v >= t_sp
                    pc = plsc.all_reduce_population_count(m)
                    pos = off + plsc.cumsum(jnp.where(m, ones, zeros)) - 1
                    okm = jnp.logical_and(m, pos < NCAND)
                    plsc.store_scatter(candv, [pos], v, mask=okm)
                    plsc.store_scatter(candi, [pos],
                                       cid_sp + iota + k * 16, mask=okm)
                    off = off + pc
                return off

            lax.fori_loop(0, nq, cand_body, zeros)

            cv = tuple(candv[pl.ds(k * 16, 16)] for k in range(NCAND // 16))

            def bis_body(_, lohi):
                lo, hi = lohi
                mid = 0.5 * (lo + hi)
                acc = zeros
                for k in range(NCAND // 16):
                    acc = acc + plsc.all_reduce_population_count(cv[k] >= mid)
                ge = acc >= TOPK
                return (jnp.where(ge, mid, lo), jnp.where(ge, hi, mid))

            tau_sp, _ = lax.fori_loop(0, 30, bis_body, (t_sp, mx_sp))

            ns_v = zeros
            zv = jnp.zeros((16,), jnp.float32)
            for k in range(NCAND // 16):
                m = cv[k] >= tau_sp
                e = jnp.exp((cv[k] - mx_sp) / TEMP)
                e = jnp.where(m, e, 0.0)
                zv = zv + e
                pc = plsc.all_reduce_population_count(m)
                pos = ns_v + plsc.cumsum(jnp.where(m, ones, zeros)) - 1
                okm = jnp.logical_and(m, pos < NG)
                plsc.store_scatter(wbuf, [pos], e, mask=okm)
                iv = candi[pl.ds(k * 16, 16)]
                plsc.store_scatter(ibuf, [pos], iv, mask=okm)
                ns_v = ns_v + pc
            z = jnp.sum(zv)
            nsel = jnp.max(jnp.minimum(ns_v, NG))

            pltpu.async_copy(es_hbm.at[ibuf], esbuf, sem).wait()

            def comb_body(j, acc):
                w_sp = _sp(wbuf, j)
                return tuple(acc[k] + w_sp * esbuf[j, pl.ds(k * 16, 16)]
                             for k in range(8))

            acc0 = tuple(jnp.zeros((16,), jnp.float32) for _ in range(8))
            acc = lax.fori_loop(0, nsel, comb_body, acc0)
            z_sp = jnp.full((16,), z, jnp.float32)
            for k in range(8):
                accbuf[rl, pl.ds(k * 16, 16)] = acc[k] / z_sp
            return 0

        lax.fori_loop(0, GR, row_body, 0)
        pltpu.sync_copy(accbuf, out_hbm.at[pl.ds(qb, GR)])
        return 0

    lax.fori_loop(0, RPW // GR, group_body, 0)


def _sc_call(sims2, cmT, t, mx, es_db):
    mesh = plsc.VectorSubcoreMesh(core_axis_name="c", subcore_axis_name="s")
    f = pl.kernel(
        _sc_body, mesh=mesh,
        out_type=jax.ShapeDtypeStruct((SLAB, D), jnp.float32),
        scratch_types=[
            pltpu.VMEM((N_CHUNK, 128), jnp.float32),
            pltpu.VMEM((RPW,), jnp.float32),
            pltpu.VMEM((RPW,), jnp.float32),
            pltpu.VMEM((16,), jnp.int32),
            pltpu.VMEM((GR, NG), jnp.int32),
            pltpu.VMEM((NG, D), jnp.float32),
            pltpu.VMEM((NCBUF,), jnp.float32),
            pltpu.VMEM((NCBUF,), jnp.int32),
            pltpu.VMEM((NG,), jnp.float32),
            pltpu.VMEM((NG,), jnp.int32),
            pltpu.VMEM((NG, D), jnp.float32),
            pltpu.VMEM((GR, D), jnp.float32),
            pltpu.SemaphoreType.DMA,
        ],
        compiler_params=pltpu.CompilerParams(needs_layout_passes=False),
    )
    return f(sims2, cmT, t, mx, es_db)


def kernel(query, en_db, es_db):
    en_pad = jnp.pad(en_db, ((0, K_PAD - K_DB), (0, 0)))
    en_norm = _l2norm_rows(en_pad, 2048)
    qn = _l2norm_rows(query, 2048)
    outs = []
    for s in range(NSLAB):
        qs = lax.slice_in_dim(qn, s * SLAB, (s + 1) * SLAB, axis=0)
        sims, cmT, t2, mx2 = _sims_call(qs, en_norm)
        sims2 = sims.reshape(SLAB * N_CHUNK, D)
        outs.append(_sc_call(sims2, cmT, t2.reshape(SLAB),
                             mx2.reshape(SLAB), es_db))
    return jnp.concatenate(outs, axis=0)


# speedup vs baseline: 27.8973x; 1.0051x over previous
import jax
import jax.numpy as jnp
from jax import lax
from jax.experimental import pallas as pl
from jax.experimental.pallas import tpu as pltpu
from jax.experimental.pallas import tpu_sc as plsc

TOPK = 70
TEMP = 0.05
Q = 4096
K_DB = 100000
D = 128
K_PAD = 100352
N_CHUNK = K_PAD // 128

BQ = 512
BK = 2048
CPB = BK // 128

NW = 32
SLAB = 512
NSLAB = Q // SLAB
RPW = SLAB // NW
GR = 16
NG = 96
NGH = NG // 2
NCAND = 128
NCBUF = NCAND + 16


def _norm_body(x_ref, o_ref):
    x = x_ref[...]
    n = jnp.sqrt(jnp.sum(x * x, axis=1, keepdims=True))
    o_ref[...] = x / jnp.maximum(n, 1e-12)


def _l2norm_rows(x, blk):
    r = x.shape[0]
    return pl.pallas_call(
        _norm_body,
        grid=(r // blk,),
        in_specs=[pl.BlockSpec((blk, D), lambda i: (i, 0))],
        out_specs=pl.BlockSpec((blk, D), lambda i: (i, 0)),
        out_shape=jax.ShapeDtypeStruct((r, D), jnp.float32),
    )(x)


def _sims_body(q_ref, e_ref, sims_ref, cm_ref, t_ref, mx_ref, cma_ref):
    j = pl.program_id(1)
    s = lax.dot_general(
        q_ref[...], e_ref[...], (((1,), (1,)), ((), ())),
        preferred_element_type=jnp.float32,
        precision=lax.Precision.DEFAULT,
    )
    col = lax.broadcasted_iota(jnp.int32, (BQ, BK), 1) + j * BK
    s = jnp.where(col < K_DB, s, -1e30)
    sims_ref[...] = s
    cm = jnp.stack(
        [jnp.max(s[:, c * 128:(c + 1) * 128], axis=1) for c in range(CPB)],
        axis=0)
    cm_ref[...] = cm
    cma_ref[pl.ds(j * CPB, CPB), :] = cm

    @pl.when(j == (K_PAD // BK) - 1)
    def _():
        cma = cma_ref[...]
        mx = jnp.max(cma, axis=0)

        def bb(_, lohi):
            lo, hi = lohi
            mid = 0.5 * (lo + hi)
            cnt = jnp.sum(jnp.where(cma >= mid[None, :], 1, 0), axis=0)
            ge = cnt >= TOPK
            return (jnp.where(ge, mid, lo), jnp.where(ge, hi, mid))

        lo, _ = lax.fori_loop(
            0, 24, bb, (jnp.full((BQ,), -1.0, jnp.float32), mx))
        t_ref[...] = lo[None, :]
        mx_ref[...] = mx[None, :]


def _sims_call(qn_slab, en_norm):
    return pl.pallas_call(
        _sims_body,
        grid=(SLAB // BQ, K_PAD // BK),
        in_specs=[
            pl.BlockSpec((BQ, D), lambda i, j: (i, 0)),
            pl.BlockSpec((BK, D), lambda i, j: (j, 0)),
        ],
        out_specs=[
            pl.BlockSpec((BQ, BK), lambda i, j: (i, j)),
            pl.BlockSpec((CPB, BQ), lambda i, j: (j, i)),
            pl.BlockSpec((1, BQ), lambda i, j: (0, i)),
            pl.BlockSpec((1, BQ), lambda i, j: (0, i)),
        ],
        out_shape=[
            jax.ShapeDtypeStruct((SLAB, K_PAD), jnp.float32),
            jax.ShapeDtypeStruct((N_CHUNK, SLAB), jnp.float32),
            jax.ShapeDtypeStruct((1, SLAB), jnp.float32),
            jax.ShapeDtypeStruct((1, SLAB), jnp.float32),
        ],
        scratch_shapes=[pltpu.VMEM((N_CHUNK, BQ), jnp.float32)],
        compiler_params=pltpu.CompilerParams(
            dimension_semantics=("arbitrary", "arbitrary"),
        ),
    )(qn_slab, en_norm)


def _sc_body(sims_hbm, cm_hbm, t_hbm, mx_hbm, es_hbm, out_hbm,
             cmbuf, tbuf, mxbuf, nqbuf, gidx2, cbufa, cbufb, candv, candi,
             wbuf, ibuf, esbuf, accbuf, sem, sema, semb):
    wid = lax.axis_index("s") * 2 + lax.axis_index("c")
    per_blk = 128 // RPW
    cb128 = (wid // per_blk) * 128
    sub = (wid % per_blk) * RPW
    w0 = cb128 + sub
    iota = lax.iota(jnp.int32, 16)
    ones = jnp.full((16,), 1, jnp.int32)
    zeros = jnp.zeros((16,), jnp.int32)

    def _sp(ref, j):
        return plsc.load_gather(ref, [jnp.full((16,), j, jnp.int32)])

    for row in range(GR):
        for sl in range(NG // 16):
            gidx2[row, pl.ds(sl * 16, 16)] = iota + sl * 16
    for sl in range(NG // 16):
        ibuf[pl.ds(sl * 16, 16)] = iota + sl * 16

    pltpu.sync_copy(cm_hbm.at[:, pl.ds(cb128, 128)], cmbuf)
    pltpu.sync_copy(t_hbm.at[pl.ds(w0, RPW)], tbuf)
    pltpu.sync_copy(mx_hbm.at[pl.ds(w0, RPW)], mxbuf)

    def group_body(g, _g):
        rb = g * GR
        rbc = sub + g * GR
        qb = w0 + g * GR
        t_vec = tbuf[pl.ds(rb, 16)]

        def chunk_body(c, nqv):
            cmrow = cmbuf[c, pl.ds(rbc, 16)]
            m = jnp.logical_and(cmrow >= t_vec, nqv < NG)
            plsc.store_scatter(gidx2, [iota, nqv],
                               (qb + iota) * N_CHUNK + c, mask=m)
            return nqv + jnp.where(m, 1, 0)

        nqv = lax.fori_loop(0, N_CHUNK, chunk_body, zeros)
        nqbuf[...] = nqv

        pltpu.async_copy(sims_hbm.at[gidx2.at[0, pl.ds(0, NGH)]],
                         cbufa, sema)

        def row_body(rl, _r):
            q = qb + rl
            nq = jnp.max(_sp(nqbuf, rl))
            t_sp = _sp(tbuf, rb + rl)
            mx_sp = _sp(mxbuf, rb + rl)

            pltpu.async_copy(sims_hbm.at[gidx2.at[rl, pl.ds(NGH, NGH)]],
                             cbufb, semb)

            neg = jnp.full((16,), -1e30, jnp.float32)
            for sl in range(NCBUF // 16):
                candv[pl.ds(sl * 16, 16)] = neg

            def make_cand_body(cbuf, jbase):
                def cand_body(j, nc_v):
                    gj = plsc.load_gather(
                        gidx2, [jnp.full((16,), rl, jnp.int32),
                                jnp.full((16,), jbase + j, jnp.int32)])
                    cid_sp = (gj - q * N_CHUNK) * 128
                    off = nc_v
                    for k in range(8):
                        v = cbuf[j, pl.ds(k * 16, 16)]
                        m = v >= t_sp
                        pc = plsc.all_reduce_population_count(m)
                        pos = off + plsc.cumsum(
                            jnp.where(m, ones, zeros)) - 1
                        okm = jnp.logical_and(m, pos < NCAND)
                        plsc.store_scatter(candv, [pos], v, mask=okm)
                        plsc.store_scatter(candi, [pos],
                                           cid_sp + iota + k * 16, mask=okm)
                        off = off + pc
                    return off
                return cand_body

            pltpu.make_async_copy(sims_hbm.at[gidx2.at[rl, pl.ds(0, NGH)]],
                                  cbufa, sema).wait()
            nc_v = lax.fori_loop(0, jnp.minimum(nq, NGH),
                                 make_cand_body(cbufa, 0), zeros)
            pltpu.make_async_copy(sims_hbm.at[gidx2.at[rl, pl.ds(NGH, NGH)]],
                                  cbufb, semb).wait()
            nc_v = lax.fori_loop(0, jnp.maximum(nq - NGH, 0),
                                 make_cand_body(cbufb, NGH), nc_v)

            @pl.when(rl < GR - 1)
            def _():
                pltpu.async_copy(
                    sims_hbm.at[gidx2.at[rl + 1, pl.ds(0, NGH)]],
                    cbufa, sema)

            cv = tuple(candv[pl.ds(k * 16, 16)] for k in range(NCAND // 16))

            def bis_body(_, lohi):
                lo, hi = lohi
                mid = 0.5 * (lo + hi)
                acc = zeros
                for k in range(NCAND // 16):
                    acc = acc + plsc.all_reduce_population_count(cv[k] >= mid)
                ge = acc >= TOPK
                return (jnp.where(ge, mid, lo), jnp.where(ge, hi, mid))

            tau_sp, _ = lax.fori_loop(0, 30, bis_body, (t_sp, mx_sp))

            zf = jnp.zeros((16,), jnp.float32)
            for sl in range(NG // 16):
                wbuf[pl.ds(sl * 16, 16)] = zf
            ns_v = zeros
            zv = jnp.zeros((16,), jnp.float32)
            for k in range(NCAND // 16):
                m = cv[k] >= tau_sp
                e = jnp.exp((cv[k] - mx_sp) / TEMP)
                e = jnp.where(m, e, 0.0)
                zv = zv + e
                pc = plsc.all_reduce_population_count(m)
                pos = ns_v + plsc.cumsum(jnp.where(m, ones, zeros)) - 1
                okm = jnp.logical_and(m, pos < NG)
                plsc.store_scatter(wbuf, [pos], e, mask=okm)
                iv = candi[pl.ds(k * 16, 16)]
                plsc.store_scatter(ibuf, [pos], iv, mask=okm)
                ns_v = ns_v + pc
            z = jnp.sum(zv)
            nsel = jnp.max(jnp.minimum(ns_v, NG))

            pltpu.async_copy(es_hbm.at[ibuf], esbuf, sem).wait()

            def comb_body(jg, acc):
                w16 = wbuf[pl.ds(jg * 16, 16)]
                acc = list(acc)
                for jj in range(16):
                    w_sp = w16.at[jnp.full((16,), jj, jnp.int32)].get(
                        mode="promise_in_bounds")
                    row = jg * 16 + jj
                    for k in range(8):
                        acc[k] = acc[k] + w_sp * esbuf[row, pl.ds(k * 16, 16)]
                return tuple(acc)

            acc0 = tuple(jnp.zeros((16,), jnp.float32) for _ in range(8))
            acc = lax.fori_loop(0, (nsel + 15) // 16, comb_body, acc0)
            z_sp = jnp.full((16,), z, jnp.float32)
            for k in range(8):
                accbuf[rl, pl.ds(k * 16, 16)] = acc[k] / z_sp
            return 0

        lax.fori_loop(0, GR, row_body, 0)
        pltpu.sync_copy(accbuf, out_hbm.at[pl.ds(qb, GR)])
        return 0

    lax.fori_loop(0, RPW // GR, group_body, 0)


def _sc_call(sims2, cmT, t, mx, es_db):
    mesh = plsc.VectorSubcoreMesh(core_axis_name="c", subcore_axis_name="s")
    f = pl.kernel(
        _sc_body, mesh=mesh,
        out_type=jax.ShapeDtypeStruct((SLAB, D), jnp.float32),
        scratch_types=[
            pltpu.VMEM((N_CHUNK, 128), jnp.float32),
            pltpu.VMEM((RPW,), jnp.float32),
            pltpu.VMEM((RPW,), jnp.float32),
            pltpu.VMEM((16,), jnp.int32),
            pltpu.VMEM((GR, NG), jnp.int32),
            pltpu.VMEM((NGH, D), jnp.float32),
            pltpu.VMEM((NGH, D), jnp.float32),
            pltpu.VMEM((NCBUF,), jnp.float32),
            pltpu.VMEM((NCBUF,), jnp.int32),
            pltpu.VMEM((NG,), jnp.float32),
            pltpu.VMEM((NG,), jnp.int32),
            pltpu.VMEM((NG, D), jnp.float32),
            pltpu.VMEM((GR, D), jnp.float32),
            pltpu.SemaphoreType.DMA,
            pltpu.SemaphoreType.DMA,
            pltpu.SemaphoreType.DMA,
        ],
        compiler_params=pltpu.CompilerParams(needs_layout_passes=False),
    )
    return f(sims2, cmT, t, mx, es_db)


def kernel(query, en_db, es_db):
    en_pad = jnp.pad(en_db, ((0, K_PAD - K_DB), (0, 0)))
    en_norm = _l2norm_rows(en_pad, 2048)
    qn = _l2norm_rows(query, 2048)
    outs = []
    for s in range(NSLAB):
        qs = lax.slice_in_dim(qn, s * SLAB, (s + 1) * SLAB, axis=0)
        sims, cmT, t2, mx2 = _sims_call(qs, en_norm)
        sims2 = sims.reshape(SLAB * N_CHUNK, D)
        outs.append(_sc_call(sims2, cmT, t2.reshape(SLAB),
                             mx2.reshape(SLAB), es_db))
    return jnp.concatenate(outs, axis=0)
